# Initial kernel scaffold; baseline (speedup 1.0000x reference)
import jax
import jax.numpy as jnp
from jax.experimental import pallas as pl


def kernel(x, edge_index, batch, stats, W1, att_src1, att_dst1, bias1, W2, att_src2, att_dst2, bias2, bn_gamma, bn_beta, fcW, fcb, muW, mub, lvW, lvb, d0W, d0b, d1W, d1b, d2W, d2b):
    raise NotImplementedError("write your pallas kernel here")


# speedup vs baseline: 1.5751x; 1.5751x over previous
import functools

import numpy as np
import jax
import jax.numpy as jnp
from jax import lax
from jax.experimental import pallas as pl
from jax.experimental.pallas import tpu as pltpu

N_NODES = 10000
N_EDGES = 320000
D_IN = 128
H_ENC = 64
LATENT = 32
H_DEC = 256
B = 200
NMAX = 50
NCOND = 7
M_PAIRS = NMAX * (NMAX - 1) // 2

_BN_SCALE = 1.0 / np.sqrt(1.0 + 1e-5)


def _pairs_scatter_matrix():
    iu0, iu1 = np.triu_indices(NMAX, k=1)
    S = np.zeros((M_PAIRS, NMAX * NMAX), dtype=np.float32)
    k = np.arange(M_PAIRS)
    S[k, iu0 * NMAX + iu1] = 1.0
    S[k, iu1 * NMAX + iu0] = 1.0
    return S


_S_MAT = _pairs_scatter_matrix()


def _dot(a, b):
    return lax.dot_general(a, b, (((a.ndim - 1,), (0,)), ((), ())),
                           preferred_element_type=jnp.float32)


def _decoder_body(out2_ref, batch_ref, stats_ref, bn_ref, fcW_ref, fcb_ref,
                  muW_ref, mub_ref, d0Wa_ref, d0Wb_ref, d0b_ref, d1W_ref,
                  d1b_ref, d2W0_ref, d2b0_ref, d2W1_ref, d2b1_ref,
                  z0_ref, z1_ref):
    out2 = out2_ref[...]
    batch = batch_ref[...]
    onehot = (batch == lax.broadcasted_iota(jnp.int32, (1, B), 1)
              ).astype(jnp.float32)
    pooled = lax.dot_general(onehot, out2, (((0,), (0,)), ((), ())),
                             precision=lax.Precision.HIGHEST,
                             preferred_element_type=jnp.float32)
    gamma = bn_ref[0:1, :]
    beta = bn_ref[1:2, :]
    outn = pooled * _BN_SCALE * gamma + beta
    xg = _dot(outn, fcW_ref[...]) + fcb_ref[...]
    mu = _dot(xg, muW_ref[...]) + mub_ref[...]
    hd = jnp.maximum(_dot(mu, d0Wa_ref[...]) + _dot(stats_ref[...], d0Wb_ref[...])
                     + d0b_ref[...], 0.0)
    hd = jnp.maximum(_dot(hd, d1W_ref[...]) + d1b_ref[...], 0.0)
    z0_ref[...] = _dot(hd, d2W0_ref[...]) + d2b0_ref[...]
    z1_ref[...] = _dot(hd, d2W1_ref[...]) + d2b1_ref[...]


@jax.jit
def _decoder(out2, batch, stats, bn2, fcW, fcb, muW, mub, d0Wa, d0Wb, d0b,
             d1W, d1b, d2W0, d2b0, d2W1, d2b1):
    return pl.pallas_call(
        _decoder_body,
        out_shape=[jax.ShapeDtypeStruct((B, M_PAIRS), jnp.float32),
                   jax.ShapeDtypeStruct((B, M_PAIRS), jnp.float32)],
    )(out2, batch, stats, bn2, fcW, fcb, muW, mub, d0Wa, d0Wb, d0b, d1W,
      d1b, d2W0, d2b0, d2W1, d2b1)


def _gumbel_tail(z0, z1):
    logits = jnp.stack([z0, z1], axis=-1)
    u = jax.random.uniform(jax.random.key(42), logits.shape,
                           minval=1e-10, maxval=1.0)
    g = -jnp.log(-jnp.log(u))
    y_soft = jax.nn.softmax(logits + g, axis=-1)
    idx = jnp.argmax(y_soft, axis=-1)
    y_hard = jax.nn.one_hot(idx, 2, dtype=y_soft.dtype)
    y = lax.stop_gradient(y_hard - y_soft) + y_soft
    xv = y[:, :, 0]
    iu = jnp.triu_indices(NMAX, k=1)
    adj = jnp.zeros((B, NMAX, NMAX), dtype=xv.dtype).at[:, iu[0], iu[1]].set(xv)
    return adj + jnp.transpose(adj, (0, 2, 1))


def kernel(x, edge_index, batch, stats, W1, att_src1, att_dst1, bias1, W2,
           att_src2, att_dst2, bias2, bn_gamma, bn_beta, fcW, fcb, muW, mub,
           lvW, lvb, d0W, d0b, d1W, d1b, d2W, d2b):
    N = x.shape[0]
    loops = jnp.arange(N, dtype=edge_index.dtype)
    ei = jnp.concatenate([edge_index, jnp.stack([loops, loops])], axis=1)
    src, dst = ei[0], ei[1]

    def gat(h_in, W, a_s, a_d, b):
        h = h_in @ W
        asrc = h @ a_s
        adst = h @ a_d
        e = jax.nn.leaky_relu(asrc[src] + adst[dst], 0.2)
        ex = jnp.exp(e)
        den = jax.ops.segment_sum(ex, dst, num_segments=N)
        acc = jax.ops.segment_sum(ex[:, None] * h[src], dst, num_segments=N)
        return acc / (den + 1e-16)[:, None] + b

    h = gat(x, W1, att_src1, att_dst1, bias1)
    out2 = gat(h, W2, att_src2, att_dst2, bias2)

    d0Wa, d0Wb = d0W[:LATENT], d0W[LATENT:]
    d2W0, d2b0 = d2W[:, 0::2], d2b[0::2]
    d2W1, d2b1 = d2W[:, 1::2], d2b[1::2]
    bn2 = jnp.stack([bn_gamma, bn_beta])

    z0, z1 = _decoder(out2, batch[:, None], stats, bn2, fcW, fcb[None, :], muW,
                      mub[None, :], d0Wa, d0Wb, d0b[None, :], d1W,
                      d1b[None, :], d2W0, d2b0[None, :], d2W1, d2b1[None, :])
    return _gumbel_tail(z0, z1)


# speedup vs baseline: 1.7386x; 1.1038x over previous
import functools

import numpy as np
import jax
import jax.numpy as jnp
from jax import lax
from jax.experimental import pallas as pl
from jax.experimental.pallas import tpu as pltpu

N_NODES = 10000
N_EDGES = 320000
D_IN = 128
H_ENC = 64
LATENT = 32
H_DEC = 256
B = 200
NMAX = 50
NCOND = 7
M_PAIRS = NMAX * (NMAX - 1) // 2

_BN_SCALE = 1.0 / np.sqrt(1.0 + 1e-5)


def _pairs_scatter_matrix():
    iu0, iu1 = np.triu_indices(NMAX, k=1)
    S = np.zeros((M_PAIRS, NMAX * NMAX), dtype=np.float32)
    k = np.arange(M_PAIRS)
    S[k, iu0 * NMAX + iu1] = 1.0
    S[k, iu1 * NMAX + iu0] = 1.0
    return S


_S_MAT = _pairs_scatter_matrix()


def _dot(a, b):
    return lax.dot_general(a, b, (((a.ndim - 1,), (0,)), ((), ())),
                           preferred_element_type=jnp.float32)


def _mm_body(a_ref, w_ref, o_ref):
    o_ref[...] = _dot(a_ref[...], w_ref[...])


def _mm(a, w):
    return pl.pallas_call(
        _mm_body,
        out_shape=jax.ShapeDtypeStruct((a.shape[0], w.shape[1]), jnp.float32),
    )(a, w)


def _decoder_body(out2_ref, batch_ref, stats_ref, bn_ref, fcW_ref, fcb_ref,
                  muW_ref, mub_ref, d0Wa_ref, d0Wb_ref, d0b_ref, d1W_ref,
                  d1b_ref, d2W0_ref, d2b0_ref, d2W1_ref, d2b1_ref,
                  z0_ref, z1_ref):
    out2 = out2_ref[...]
    batch = batch_ref[...]
    onehot = (batch == lax.broadcasted_iota(jnp.int32, (1, B), 1)
              ).astype(jnp.float32)
    pooled = lax.dot_general(onehot, out2, (((0,), (0,)), ((), ())),
                             precision=lax.Precision.HIGHEST,
                             preferred_element_type=jnp.float32)
    gamma = bn_ref[0:1, :]
    beta = bn_ref[1:2, :]
    outn = pooled * _BN_SCALE * gamma + beta
    xg = _dot(outn, fcW_ref[...]) + fcb_ref[...]
    mu = _dot(xg, muW_ref[...]) + mub_ref[...]
    hd = jnp.maximum(_dot(mu, d0Wa_ref[...]) + _dot(stats_ref[...], d0Wb_ref[...])
                     + d0b_ref[...], 0.0)
    hd = jnp.maximum(_dot(hd, d1W_ref[...]) + d1b_ref[...], 0.0)
    z0_ref[...] = _dot(hd, d2W0_ref[...]) + d2b0_ref[...]
    z1_ref[...] = _dot(hd, d2W1_ref[...]) + d2b1_ref[...]


@jax.jit
def _decoder(out2, batch, stats, bn2, fcW, fcb, muW, mub, d0Wa, d0Wb, d0b,
             d1W, d1b, d2W0, d2b0, d2W1, d2b1):
    return pl.pallas_call(
        _decoder_body,
        out_shape=[jax.ShapeDtypeStruct((B, M_PAIRS), jnp.float32),
                   jax.ShapeDtypeStruct((B, M_PAIRS), jnp.float32)],
    )(out2, batch, stats, bn2, fcW, fcb, muW, mub, d0Wa, d0Wb, d0b, d1W,
      d1b, d2W0, d2b0, d2W1, d2b1)


def _gumbel_tail(z0, z1):
    logits = jnp.stack([z0, z1], axis=-1)
    u = jax.random.uniform(jax.random.key(42), logits.shape,
                           minval=1e-10, maxval=1.0)
    g = -jnp.log(-jnp.log(u))
    y_soft = jax.nn.softmax(logits + g, axis=-1)
    idx = jnp.argmax(y_soft, axis=-1)
    y_hard = jax.nn.one_hot(idx, 2, dtype=y_soft.dtype)
    y = lax.stop_gradient(y_hard - y_soft) + y_soft
    xv = y[:, :, 0]
    iu = jnp.triu_indices(NMAX, k=1)
    adj = jnp.zeros((B, NMAX, NMAX), dtype=xv.dtype).at[:, iu[0], iu[1]].set(xv)
    return adj + jnp.transpose(adj, (0, 2, 1))


def kernel(x, edge_index, batch, stats, W1, att_src1, att_dst1, bias1, W2,
           att_src2, att_dst2, bias2, bn_gamma, bn_beta, fcW, fcb, muW, mub,
           lvW, lvb, d0W, d0b, d1W, d1b, d2W, d2b):
    N = x.shape[0]
    loops = jnp.arange(N, dtype=edge_index.dtype)
    ei = jnp.concatenate([edge_index, jnp.stack([loops, loops])], axis=1)
    src, dst = ei[0], ei[1]

    def gat(h_in, W, a_s, a_d, b):
        h = _mm(h_in, W)
        asrc = h @ a_s
        adst = h @ a_d
        e = jax.nn.leaky_relu(asrc[src] + adst[dst], 0.2)
        ex = jnp.exp(e)
        den = jax.ops.segment_sum(ex, dst, num_segments=N)
        acc = jax.ops.segment_sum(ex[:, None] * h[src], dst, num_segments=N)
        return acc / (den + 1e-16)[:, None] + b

    h = gat(x, W1, att_src1, att_dst1, bias1)
    out2 = gat(h, W2, att_src2, att_dst2, bias2)

    d0Wa, d0Wb = d0W[:LATENT], d0W[LATENT:]
    d2W0, d2b0 = d2W[:, 0::2], d2b[0::2]
    d2W1, d2b1 = d2W[:, 1::2], d2b[1::2]
    bn2 = jnp.stack([bn_gamma, bn_beta])

    z0, z1 = _decoder(out2, batch[:, None], stats, bn2, fcW, fcb[None, :], muW,
                      mub[None, :], d0Wa, d0Wb, d0b[None, :], d1W,
                      d1b[None, :], d2W0, d2b0[None, :], d2W1, d2b1[None, :])
    return _gumbel_tail(z0, z1)
